# baseline (device time: 81788 ns/iter reference)
import jax
import jax.numpy as jnp
from jax import lax
from jax.experimental import pallas as pl
from jax.experimental.pallas import tpu as pltpu

N = 8
T = 1024
D = 512
CAP = 192
CH = 16

_MESH = pl.DeviceIdType.MESH


def _a2av_body(slots_ref, cnt_ref, out_ref, cnt_out_ref,
               send_c, recv_c, send_d, recv_d):
    me = lax.axis_index("i")

    bar = pltpu.get_barrier_semaphore()
    for k in range(1, N):
        peer = lax.rem(me + k, N)
        pl.semaphore_signal(bar, 1, device_id=(peer,), device_id_type=_MESH)
    pl.semaphore_wait(bar, N - 1)

    cdescs = []
    for k in range(1, N):
        peer = lax.rem(me + k, N)
        c = pltpu.make_async_remote_copy(
            src_ref=cnt_ref.at[peer],
            dst_ref=cnt_out_ref.at[me],
            send_sem=send_c, recv_sem=recv_c,
            device_id=(peer,), device_id_type=_MESH)
        c.start()
        cdescs.append(c)

    total_send = jnp.int32(0)
    for j in range(N):
        nch = lax.div(cnt_ref[j, 0, 0] + (CH - 1), CH)
        nch_eff = jnp.where(jnp.int32(j) == me, 0, nch)

        def send_chunk(n, carry, j=j):
            d = pltpu.make_async_remote_copy(
                src_ref=slots_ref.at[j, pl.ds(n * CH, CH)],
                dst_ref=out_ref.at[me, pl.ds(n * CH, CH)],
                send_sem=send_d, recv_sem=recv_d,
                device_id=(j,), device_id_type=_MESH)
            d.start()
            return carry

        lax.fori_loop(0, nch_eff, send_chunk, 0)
        total_send = total_send + nch_eff

    out_ref[me] = slots_ref[me]
    cnt_out_ref[me] = cnt_ref[me]

    for c in cdescs:
        c.wait_recv()

    total_recv = jnp.int32(0)
    for i in range(N):
        nch = lax.div(cnt_out_ref[i, 0, 0] + (CH - 1), CH)
        total_recv = total_recv + jnp.where(jnp.int32(i) == me, 0, nch)

    def _chunk_desc():
        return pltpu.make_async_remote_copy(
            src_ref=slots_ref.at[0, pl.ds(0, CH)],
            dst_ref=out_ref.at[0, pl.ds(0, CH)],
            send_sem=send_d, recv_sem=recv_d,
            device_id=(me,), device_id_type=_MESH)

    def wrecv(n, carry):
        _chunk_desc().wait_recv()
        return carry

    lax.fori_loop(0, total_recv, wrecv, 0)

    for c in cdescs:
        c.wait_send()

    def wsend(n, carry):
        _chunk_desc().wait_send()
        return carry

    lax.fori_loop(0, total_send, wsend, 0)


def _a2av_pallas(slots, cnt_rows):
    return pl.pallas_call(
        _a2av_body,
        out_shape=(
            jax.ShapeDtypeStruct((N, CAP, D), jnp.bfloat16),
            jax.ShapeDtypeStruct((N, 1, 128), jnp.int32),
        ),
        in_specs=[
            pl.BlockSpec(memory_space=pltpu.VMEM),
            pl.BlockSpec(memory_space=pltpu.VMEM),
        ],
        out_specs=(
            pl.BlockSpec(memory_space=pltpu.VMEM),
            pl.BlockSpec(memory_space=pltpu.VMEM),
        ),
        scratch_shapes=[
            pltpu.SemaphoreType.DMA,
            pltpu.SemaphoreType.DMA,
            pltpu.SemaphoreType.DMA,
            pltpu.SemaphoreType.DMA,
        ],
        compiler_params=pltpu.CompilerParams(collective_id=0),
    )(slots, cnt_rows)


def kernel(x, dest):
    dest = dest.astype(jnp.int32)

    oh = (dest[:, None] == jnp.arange(N, dtype=jnp.int32)[None, :])
    ohi = oh.astype(jnp.int32)
    counts = jnp.sum(ohi, axis=0)
    pos = jnp.sum(jnp.cumsum(ohi, axis=0) * ohi, axis=1) - 1
    slots = jnp.zeros((N * CAP, D), jnp.bfloat16)
    slots = slots.at[dest * CAP + pos].set(x.astype(jnp.bfloat16))
    slots = slots.reshape(N, CAP, D)
    cnt_rows = jnp.zeros((N, 1, 128), jnp.int32).at[:, 0, 0].set(counts)

    staging, cnt_in = _a2av_pallas(slots, cnt_rows)

    cnt_col = cnt_in[:, 0, 0]
    starts = jnp.concatenate(
        [jnp.zeros((1,), jnp.int32), jnp.cumsum(cnt_col)[:-1]]
    ).astype(jnp.int32)
    r = jnp.arange(T, dtype=jnp.int32)
    src = jnp.searchsorted(starts, r, side="right").astype(jnp.int32) - 1
    off = r - starts[src]
    return staging.reshape(N * CAP, D)[src * CAP + off]


# device time: 76425 ns/iter; 1.0702x vs baseline; 1.0702x over previous
import jax
import jax.numpy as jnp
from jax import lax
from jax.experimental import pallas as pl
from jax.experimental.pallas import tpu as pltpu

N = 8
T = 1024
D = 512
CAP = 192
CH = 16

_MESH = pl.DeviceIdType.MESH


def _a2av_body(slots_ref, cnt_ref, out_ref, cnt_out_ref,
               send_c, recv_c, send_d, recv_d):
    me = lax.axis_index("i")

    bar = pltpu.get_barrier_semaphore()
    for k in range(1, N):
        peer = lax.rem(me + k, N)
        pl.semaphore_signal(bar, 1, device_id=(peer,), device_id_type=_MESH)
    pl.semaphore_wait(bar, N - 1)

    cdescs = []
    for k in range(1, N):
        peer = lax.rem(me + k, N)
        c = pltpu.make_async_remote_copy(
            src_ref=cnt_ref.at[peer],
            dst_ref=cnt_out_ref.at[me],
            send_sem=send_c, recv_sem=recv_c,
            device_id=(peer,), device_id_type=_MESH)
        c.start()
        cdescs.append(c)

    total_send = jnp.int32(0)
    for j in range(N):
        nch = lax.div(cnt_ref[j, 0, 0] + (CH - 1), CH)
        nch_eff = jnp.where(jnp.int32(j) == me, 0, nch)

        def send_chunk(n, carry, j=j):
            d = pltpu.make_async_remote_copy(
                src_ref=slots_ref.at[j, pl.ds(n * CH, CH)],
                dst_ref=out_ref.at[me, pl.ds(n * CH, CH)],
                send_sem=send_d, recv_sem=recv_d,
                device_id=(j,), device_id_type=_MESH)
            d.start()
            return carry

        lax.fori_loop(0, nch_eff, send_chunk, 0)
        total_send = total_send + nch_eff

    out_ref[me] = slots_ref[me]
    cnt_out_ref[me] = cnt_ref[me]

    for c in cdescs:
        c.wait_recv()

    total_recv = jnp.int32(0)
    for i in range(N):
        nch = lax.div(cnt_out_ref[i, 0, 0] + (CH - 1), CH)
        total_recv = total_recv + jnp.where(jnp.int32(i) == me, 0, nch)

    def _chunk_desc():
        return pltpu.make_async_remote_copy(
            src_ref=slots_ref.at[0, pl.ds(0, CH)],
            dst_ref=out_ref.at[0, pl.ds(0, CH)],
            send_sem=send_d, recv_sem=recv_d,
            device_id=(me,), device_id_type=_MESH)

    def wrecv(n, carry):
        _chunk_desc().wait_recv()
        return carry

    lax.fori_loop(0, total_recv, wrecv, 0)

    for c in cdescs:
        c.wait_send()

    def wsend(n, carry):
        _chunk_desc().wait_send()
        return carry

    lax.fori_loop(0, total_send, wsend, 0)


def _a2av_pallas(slots, cnt_rows):
    return pl.pallas_call(
        _a2av_body,
        out_shape=(
            jax.ShapeDtypeStruct((N, CAP, D), jnp.bfloat16),
            jax.ShapeDtypeStruct((N, 1, 128), jnp.int32),
        ),
        in_specs=[
            pl.BlockSpec(memory_space=pltpu.VMEM),
            pl.BlockSpec(memory_space=pltpu.VMEM),
        ],
        out_specs=(
            pl.BlockSpec(memory_space=pltpu.VMEM),
            pl.BlockSpec(memory_space=pltpu.VMEM),
        ),
        scratch_shapes=[
            pltpu.SemaphoreType.DMA,
            pltpu.SemaphoreType.DMA,
            pltpu.SemaphoreType.DMA,
            pltpu.SemaphoreType.DMA,
        ],
        compiler_params=pltpu.CompilerParams(collective_id=0),
    )(slots, cnt_rows)


def kernel(x, dest):
    dest = dest.astype(jnp.int32)

    oh = (dest[:, None] == jnp.arange(N, dtype=jnp.int32)[None, :])
    ohi = oh.astype(jnp.int32)
    counts = jnp.sum(ohi, axis=0)
    pos = jnp.sum(jnp.cumsum(ohi, axis=0) * ohi, axis=1) - 1
    slot_id = dest * CAP + pos
    perm = (slot_id[None, :] == jnp.arange(N * CAP, dtype=jnp.int32)[:, None])
    slots = jnp.dot(
        perm.astype(jnp.bfloat16), x.astype(jnp.bfloat16),
        preferred_element_type=jnp.bfloat16,
    ).reshape(N, CAP, D)
    cnt_rows = jnp.zeros((N, 1, 128), jnp.int32).at[:, 0, 0].set(counts)

    staging, cnt_in = _a2av_pallas(slots, cnt_rows)

    cnt_col = cnt_in[:, 0, 0]
    starts = jnp.concatenate(
        [jnp.zeros((1,), jnp.int32), jnp.cumsum(cnt_col)[:-1]]
    ).astype(jnp.int32)
    r = jnp.arange(T, dtype=jnp.int32)
    src = jnp.searchsorted(starts, r, side="right").astype(jnp.int32) - 1
    off = r - starts[src]
    sel = src * CAP + off
    pick = (sel[:, None] == jnp.arange(N * CAP, dtype=jnp.int32)[None, :])
    return jnp.dot(
        pick.astype(jnp.bfloat16), staging.reshape(N * CAP, D),
        preferred_element_type=jnp.bfloat16,
    )


# device time: 23433 ns/iter; 3.4903x vs baseline; 3.2614x over previous
import jax
import jax.numpy as jnp
from jax import lax
from jax.experimental import pallas as pl
from jax.experimental.pallas import tpu as pltpu

N = 8
T = 1024
D = 512
CAP = 192
CH = 16

_MESH = pl.DeviceIdType.MESH


def _a2av_body(x_ref, slot_ref, cnt_ref, out_ref,
               slots_ref, staging_ref, cnt_out_ref,
               send_c, recv_c, send_d, recv_d):
    me = lax.axis_index("i")

    staging_ref[...] = jnp.zeros_like(staging_ref)

    bar = pltpu.get_barrier_semaphore()
    for k in range(1, N):
        peer = lax.rem(me + k, N)
        pl.semaphore_signal(bar, 1, device_id=(peer,), device_id_type=_MESH)

    xb = x_ref[...]
    srow = slot_ref[...]
    for j in range(N):
        pj = (lax.broadcasted_iota(jnp.int32, (CAP, T), 0) + (j * CAP)
              == srow).astype(jnp.bfloat16)
        slots_ref[j] = jnp.dot(
            pj, xb, preferred_element_type=jnp.float32
        ).astype(jnp.bfloat16)

    pl.semaphore_wait(bar, N - 1)

    cdescs = []
    for k in range(1, N):
        peer = lax.rem(me + k, N)
        c = pltpu.make_async_remote_copy(
            src_ref=cnt_ref.at[peer],
            dst_ref=cnt_out_ref.at[me],
            send_sem=send_c, recv_sem=recv_c,
            device_id=(peer,), device_id_type=_MESH)
        c.start()
        cdescs.append(c)

    total_send = jnp.int32(0)
    for j in range(N):
        nch = lax.div(cnt_ref[j, 0, 0] + (CH - 1), CH)
        nch_eff = jnp.where(jnp.int32(j) == me, 0, nch)

        def send_chunk(n, carry, j=j):
            d = pltpu.make_async_remote_copy(
                src_ref=slots_ref.at[j, pl.ds(n * CH, CH)],
                dst_ref=staging_ref.at[me, pl.ds(n * CH, CH)],
                send_sem=send_d, recv_sem=recv_d,
                device_id=(j,), device_id_type=_MESH)
            d.start()
            return carry

        lax.fori_loop(0, nch_eff, send_chunk, 0)
        total_send = total_send + nch_eff

    staging_ref[me] = slots_ref[me]
    cnt_out_ref[me] = cnt_ref[me]

    for c in cdescs:
        c.wait_recv()

    total_recv = jnp.int32(0)
    start = jnp.int32(0)
    r = lax.broadcasted_iota(jnp.int32, (T, 1), 0)
    src_cnt = jnp.zeros((T, 1), jnp.int32)
    start_of_src = jnp.zeros((T, 1), jnp.int32)
    for i in range(N):
        ci = cnt_out_ref[i, 0, 0]
        nch = lax.div(ci + (CH - 1), CH)
        total_recv = total_recv + jnp.where(jnp.int32(i) == me, 0, nch)
        le = (r >= start).astype(jnp.int32)
        src_cnt = src_cnt + le
        start_of_src = jnp.maximum(start_of_src, le * start)
        start = start + ci
    sel = (src_cnt - 1) * CAP + r - start_of_src

    def _chunk_desc():
        return pltpu.make_async_remote_copy(
            src_ref=slots_ref.at[0, pl.ds(0, CH)],
            dst_ref=staging_ref.at[0, pl.ds(0, CH)],
            send_sem=send_d, recv_sem=recv_d,
            device_id=(me,), device_id_type=_MESH)

    def wrecv(n, carry):
        _chunk_desc().wait_recv()
        return carry

    lax.fori_loop(0, total_recv, wrecv, 0)

    acc = jnp.zeros((T, D), jnp.float32)
    for j in range(N):
        pj = (sel == lax.broadcasted_iota(jnp.int32, (T, CAP), 1)
              + (j * CAP)).astype(jnp.bfloat16)
        acc = acc + jnp.dot(
            pj, staging_ref[j], preferred_element_type=jnp.float32
        )
    out_ref[...] = acc.astype(jnp.bfloat16)

    for c in cdescs:
        c.wait_send()

    def wsend(n, carry):
        _chunk_desc().wait_send()
        return carry

    lax.fori_loop(0, total_send, wsend, 0)


def _a2av_pallas(xb, slot_row, cnt_rows):
    return pl.pallas_call(
        _a2av_body,
        out_shape=jax.ShapeDtypeStruct((T, D), jnp.bfloat16),
        in_specs=[
            pl.BlockSpec(memory_space=pltpu.VMEM),
            pl.BlockSpec(memory_space=pltpu.VMEM),
            pl.BlockSpec(memory_space=pltpu.VMEM),
        ],
        out_specs=pl.BlockSpec(memory_space=pltpu.VMEM),
        scratch_shapes=[
            pltpu.VMEM((N, CAP, D), jnp.bfloat16),
            pltpu.VMEM((N, CAP, D), jnp.bfloat16),
            pltpu.VMEM((N, 1, 128), jnp.int32),
            pltpu.SemaphoreType.DMA,
            pltpu.SemaphoreType.DMA,
            pltpu.SemaphoreType.DMA,
            pltpu.SemaphoreType.DMA,
        ],
        compiler_params=pltpu.CompilerParams(collective_id=0),
    )(xb, slot_row, cnt_rows)


def kernel(x, dest):
    dest = dest.astype(jnp.int32)

    oh = (dest[:, None] == jnp.arange(N, dtype=jnp.int32)[None, :])
    ohi = oh.astype(jnp.int32)
    counts = jnp.sum(ohi, axis=0)
    pos = jnp.sum(jnp.cumsum(ohi, axis=0) * ohi, axis=1) - 1
    slot_row = (dest * CAP + pos)[None, :]
    cnt_rows = jnp.zeros((N, 1, 128), jnp.int32).at[:, 0, 0].set(counts)

    return _a2av_pallas(x.astype(jnp.bfloat16), slot_row, cnt_rows)


# device time: 19061 ns/iter; 4.2909x vs baseline; 1.2294x over previous
import jax
import jax.numpy as jnp
from jax import lax
from jax.experimental import pallas as pl
from jax.experimental.pallas import tpu as pltpu

N = 8
T = 1024
D = 512
CAP = 192
CH = 16

_MESH = pl.DeviceIdType.MESH


def _a2av_body(x_ref, dest_ref, out_ref,
               slots_ref, staging_ref, cnt_ref, cnt_out_ref,
               send_c, recv_c, send_d, recv_d):
    me = lax.axis_index("i")

    staging_ref[...] = jnp.zeros_like(staging_ref)

    bar = pltpu.get_barrier_semaphore()
    for k in range(1, N):
        peer = lax.rem(me + k, N)
        pl.semaphore_signal(bar, 1, device_id=(peer,), device_id_type=_MESH)

    drow = dest_ref[...]
    oh = (lax.broadcasted_iota(jnp.int32, (N, T), 0) == drow)
    ohb = oh.astype(jnp.bfloat16)
    counts = jnp.sum(oh.astype(jnp.int32), axis=1, keepdims=True)
    tri = (lax.broadcasted_iota(jnp.int32, (T, T), 0)
           < lax.broadcasted_iota(jnp.int32, (T, T), 1)).astype(jnp.bfloat16)
    before = jnp.dot(ohb, tri, preferred_element_type=jnp.float32)
    pos = jnp.sum(before * ohb, axis=0, keepdims=True).astype(jnp.int32)
    srow = drow * CAP + pos

    cnt_ref[:, 0, :] = jnp.where(
        lax.broadcasted_iota(jnp.int32, (N, 128), 1) == 0, counts, 0)

    xb = x_ref[...].astype(jnp.bfloat16)
    for j in range(N):
        pj = (lax.broadcasted_iota(jnp.int32, (CAP, T), 0) + (j * CAP)
              == srow).astype(jnp.bfloat16)
        slots_ref[j] = jnp.dot(
            pj, xb, preferred_element_type=jnp.float32
        ).astype(jnp.bfloat16)

    pl.semaphore_wait(bar, N - 1)

    cdescs = []
    for k in range(1, N):
        peer = lax.rem(me + k, N)
        c = pltpu.make_async_remote_copy(
            src_ref=cnt_ref.at[peer],
            dst_ref=cnt_out_ref.at[me],
            send_sem=send_c, recv_sem=recv_c,
            device_id=(peer,), device_id_type=_MESH)
        c.start()
        cdescs.append(c)

    total_send = jnp.int32(0)
    for j in range(N):
        nch = lax.div(cnt_ref[j, 0, 0] + (CH - 1), CH)
        nch_eff = jnp.where(jnp.int32(j) == me, 0, nch)

        def send_chunk(n, carry, j=j):
            d = pltpu.make_async_remote_copy(
                src_ref=slots_ref.at[j, pl.ds(n * CH, CH)],
                dst_ref=staging_ref.at[me, pl.ds(n * CH, CH)],
                send_sem=send_d, recv_sem=recv_d,
                device_id=(j,), device_id_type=_MESH)
            d.start()
            return carry

        lax.fori_loop(0, nch_eff, send_chunk, 0)
        total_send = total_send + nch_eff

    staging_ref[me] = slots_ref[me]
    cnt_out_ref[me] = cnt_ref[me]

    for c in cdescs:
        c.wait_recv()

    total_recv = jnp.int32(0)
    start = jnp.int32(0)
    r = lax.broadcasted_iota(jnp.int32, (T, 1), 0)
    src_cnt = jnp.zeros((T, 1), jnp.int32)
    start_of_src = jnp.zeros((T, 1), jnp.int32)
    for i in range(N):
        ci = cnt_out_ref[i, 0, 0]
        nch = lax.div(ci + (CH - 1), CH)
        total_recv = total_recv + jnp.where(jnp.int32(i) == me, 0, nch)
        le = (r >= start).astype(jnp.int32)
        src_cnt = src_cnt + le
        start_of_src = jnp.maximum(start_of_src, le * start)
        start = start + ci
    sel = (src_cnt - 1) * CAP + r - start_of_src

    def _chunk_desc():
        return pltpu.make_async_remote_copy(
            src_ref=slots_ref.at[0, pl.ds(0, CH)],
            dst_ref=staging_ref.at[0, pl.ds(0, CH)],
            send_sem=send_d, recv_sem=recv_d,
            device_id=(me,), device_id_type=_MESH)

    def wrecv(n, carry):
        _chunk_desc().wait_recv()
        return carry

    lax.fori_loop(0, total_recv, wrecv, 0)

    acc = jnp.zeros((T, D), jnp.float32)
    for j in range(N):
        pj = (sel == lax.broadcasted_iota(jnp.int32, (T, CAP), 1)
              + (j * CAP)).astype(jnp.bfloat16)
        acc = acc + jnp.dot(
            pj, staging_ref[j], preferred_element_type=jnp.float32
        )
    out_ref[...] = acc.astype(jnp.bfloat16)

    for c in cdescs:
        c.wait_send()

    def wsend(n, carry):
        _chunk_desc().wait_send()
        return carry

    lax.fori_loop(0, total_send, wsend, 0)


def _a2av_pallas(x, dest_row):
    return pl.pallas_call(
        _a2av_body,
        out_shape=jax.ShapeDtypeStruct((T, D), jnp.bfloat16),
        in_specs=[
            pl.BlockSpec(memory_space=pltpu.VMEM),
            pl.BlockSpec(memory_space=pltpu.VMEM),
        ],
        out_specs=pl.BlockSpec(memory_space=pltpu.VMEM),
        scratch_shapes=[
            pltpu.VMEM((N, CAP, D), jnp.bfloat16),
            pltpu.VMEM((N, CAP, D), jnp.bfloat16),
            pltpu.VMEM((N, 1, 128), jnp.int32),
            pltpu.VMEM((N, 1, 128), jnp.int32),
            pltpu.SemaphoreType.DMA,
            pltpu.SemaphoreType.DMA,
            pltpu.SemaphoreType.DMA,
            pltpu.SemaphoreType.DMA,
        ],
        compiler_params=pltpu.CompilerParams(collective_id=0),
    )(x, dest_row)


def kernel(x, dest):
    return _a2av_pallas(x, dest.astype(jnp.int32)[None, :])
